# baseline (speedup 1.0000x reference)
import jax
import jax.numpy as jnp
from jax import lax
from jax.experimental import pallas as pl
from jax.experimental.pallas import tpu as pltpu
from jax.experimental.pallas import tpu_sc as plsc

_N = 10000
_H2 = 64
_E = 160000

_NC = 2
_NS = 16
_NW = _NC * _NS
_CHUNK = 128
_NCHUNK = 40
_EPAD = _NW * _NCHUNK * _CHUNK

_BM = 400


def _rows_mm_kernel(x_ref, w_ref, o_ref):
    o_ref[...] = jnp.dot(
        x_ref[...], w_ref[...],
        preferred_element_type=jnp.float32).astype(o_ref.dtype)


def _rows_mm(x, w, bm, out_dtype=jnp.float32):
    n, k = x.shape
    m = w.shape[1]
    return pl.pallas_call(
        _rows_mm_kernel,
        grid=(n // bm,),
        in_specs=[
            pl.BlockSpec((bm, k), lambda i: (i, 0)),
            pl.BlockSpec((k, m), lambda i: (0, 0)),
        ],
        out_specs=pl.BlockSpec((bm, m), lambda i: (i, 0)),
        out_shape=jax.ShapeDtypeStruct((n, m), out_dtype),
        compiler_params=pltpu.CompilerParams(
            dimension_semantics=("parallel",),
        ),
    )(x, w)


_BA = 200
_BT = 1000


def _h1q_kernel(a_ref, p_ref, w23_ref, q_ref, abf_ref):
    abf = a_ref[...].astype(jnp.bfloat16)
    abf_ref[...] = abf
    h1 = jnp.maximum(
        jnp.dot(abf, p_ref[...], preferred_element_type=jnp.float32), 0.0)
    q_ref[...] = jnp.dot(
        h1.astype(jnp.bfloat16), w23_ref[...],
        preferred_element_type=jnp.float32).astype(jnp.bfloat16)


def _h1q_mm(adj, p, w23):
    n = adj.shape[0]
    m = p.shape[1]
    mq = w23.shape[1]
    return pl.pallas_call(
        _h1q_kernel,
        grid=(n // _BA,),
        in_specs=[
            pl.BlockSpec((_BA, n), lambda i: (i, 0)),
            pl.BlockSpec((n, m), lambda i: (0, 0)),
            pl.BlockSpec((m, mq), lambda i: (0, 0)),
        ],
        out_specs=(
            pl.BlockSpec((_BA, mq), lambda i: (i, 0)),
            pl.BlockSpec((_BA, n), lambda i: (i, 0)),
        ),
        out_shape=(
            jax.ShapeDtypeStruct((n, mq), jnp.bfloat16),
            jax.ShapeDtypeStruct((n, n), jnp.bfloat16),
        ),
        compiler_params=pltpu.CompilerParams(
            dimension_semantics=("parallel",),
        ),
    )(adj, p, w23)


def _tail_kernel(a_ref, q_ref, eps_ref, a1w_ref, a1b_ref, a2w_ref, a2b_ref,
                 mu_ref, lv_ref, z_ref, adv_ref):
    acc = jnp.dot(a_ref[...], q_ref[...], preferred_element_type=jnp.float32)
    mu = acc[:, :_H2]
    lv = acc[:, _H2:]
    z = eps_ref[...] * jnp.exp(lv) + mu
    mu_ref[...] = mu
    lv_ref[...] = lv
    z_ref[...] = z
    hidden = jnp.maximum(
        jnp.dot(z, a1w_ref[...], preferred_element_type=jnp.float32)
        + a1b_ref[...], 0.0)
    adv_ref[...] = (
        jnp.dot(hidden, a2w_ref[...], preferred_element_type=jnp.float32)
        + a2b_ref[...])


def _tail_mm(adj, q, eps, a1w, a1b, a2w, a2b):
    n = adj.shape[0]
    out_shapes = (
        jax.ShapeDtypeStruct((n, _H2), jnp.float32),
        jax.ShapeDtypeStruct((n, _H2), jnp.float32),
        jax.ShapeDtypeStruct((n, _H2), jnp.float32),
        jax.ShapeDtypeStruct((n, 1), jnp.float32),
    )
    out_spec = pl.BlockSpec((_BT, _H2), lambda i: (i, 0))
    return pl.pallas_call(
        _tail_kernel,
        grid=(n // _BT,),
        in_specs=[
            pl.BlockSpec((_BT, n), lambda i: (i, 0)),
            pl.BlockSpec((n, 2 * _H2), lambda i: (0, 0)),
            pl.BlockSpec((_BT, _H2), lambda i: (i, 0)),
            pl.BlockSpec((_H2, _H2), lambda i: (0, 0)),
            pl.BlockSpec((1, _H2), lambda i: (0, 0)),
            pl.BlockSpec((_H2, 1), lambda i: (0, 0)),
            pl.BlockSpec((1, 1), lambda i: (0, 0)),
        ],
        out_specs=(out_spec, out_spec, out_spec,
                   pl.BlockSpec((_BT, 1), lambda i: (i, 0))),
        out_shape=out_shapes,
        compiler_params=pltpu.CompilerParams(
            dimension_semantics=("parallel",),
        ),
    )(adj, q, eps, a1w, a1b, a2w, a2b)


def _recov_kernel(zi_ref, zt_ref, o_ref):
    o_ref[...] = jnp.dot(zi_ref[...].astype(jnp.bfloat16),
                         zt_ref[...].astype(jnp.bfloat16),
                         preferred_element_type=jnp.float32)


def _recov_mm(z, zt):
    n = z.shape[0]
    return pl.pallas_call(
        _recov_kernel,
        grid=(n // _BM,),
        in_specs=[
            pl.BlockSpec((_BM, _H2), lambda i: (i, 0)),
            pl.BlockSpec((_H2, n), lambda i: (0, 0)),
        ],
        out_specs=pl.BlockSpec((_BM, n), lambda i: (i, 0)),
        out_shape=jax.ShapeDtypeStruct((n, n), jnp.float32),
        compiler_params=pltpu.CompilerParams(
            dimension_semantics=("parallel",),
        ),
    )(z, zt)


_DEPTH = 8


def _link_body(recov_hbm, idx_hbm, out_hbm, idxs, vals, sem_g):
    wid = lax.axis_index("s") * _NC + lax.axis_index("c")

    pltpu.sync_copy(idx_hbm.at[wid], idxs)

    def chunk(c, _):
        pltpu.async_copy(recov_hbm.at[idxs.at[c]], vals.at[c], sem_g)

        @pl.when(c >= _DEPTH)
        def _():
            pltpu.make_async_copy(
                recov_hbm.at[idxs.at[0]], vals.at[0], sem_g).wait()

        return 0

    lax.fori_loop(0, _NCHUNK, chunk, 0)
    for _ in range(_DEPTH):
        pltpu.make_async_copy(recov_hbm.at[idxs.at[0]], vals.at[0], sem_g).wait()
    pltpu.sync_copy(vals, out_hbm.at[wid])


def _link_preds_sc(recov_flat, idxr):
    mesh = plsc.VectorSubcoreMesh(
        core_axis_name="c", subcore_axis_name="s",
        num_cores=_NC, num_subcores=_NS)
    k = pl.kernel(
        _link_body,
        out_type=jax.ShapeDtypeStruct((_NW, _NCHUNK, _CHUNK), jnp.float32),
        mesh=mesh,
        scratch_types=[
            pltpu.VMEM((_NCHUNK, _CHUNK), jnp.int32),
            pltpu.VMEM((_NCHUNK, _CHUNK), jnp.float32),
            pltpu.SemaphoreType.DMA,
        ],
        compiler_params=pltpu.CompilerParams(needs_layout_passes=False),
    )
    return k(recov_flat, idxr)


def kernel(feats, adj, edges, W1, W2, W3, A1w, A1b, A2w, A2b, eps):
    w23 = jnp.concatenate([W2, W3], axis=1).astype(jnp.bfloat16)
    p = _rows_mm(feats, W1, 2000, out_dtype=jnp.bfloat16)
    q, adj_bf = _h1q_mm(adj, p, w23)
    mu, logvar, z, adv_preds = _tail_mm(
        adj_bf, q, eps, A1w, A1b.reshape(1, _H2), A2w, A2b.reshape(1, 1))

    recov = _recov_mm(z, z.T)

    pad = _EPAD - _E
    flat = edges[:, 0] * _N + edges[:, 1]
    flat = jnp.concatenate([flat, jnp.zeros((pad,), flat.dtype)])
    idxr = flat.astype(jnp.int32).reshape(_NW, _NCHUNK, _CHUNK)
    link = _link_preds_sc(recov.reshape(-1), idxr).reshape(-1)[:_E]

    return (recov, mu, logvar, link, adv_preds)


# speedup vs baseline: 1.0143x; 1.0143x over previous
import jax
import jax.numpy as jnp
from jax import lax
from jax.experimental import pallas as pl
from jax.experimental.pallas import tpu as pltpu
from jax.experimental.pallas import tpu_sc as plsc

_N = 10000
_H2 = 64
_E = 160000

_NC = 2
_NS = 16
_NW = _NC * _NS
_CHUNK = 128
_NCHUNK = 40
_EPAD = _NW * _NCHUNK * _CHUNK

_BM = 400


def _rows_mm_kernel(x_ref, w_ref, o_ref):
    o_ref[...] = jnp.dot(
        x_ref[...], w_ref[...],
        preferred_element_type=jnp.float32).astype(o_ref.dtype)


def _rows_mm(x, w, bm, out_dtype=jnp.float32):
    n, k = x.shape
    m = w.shape[1]
    return pl.pallas_call(
        _rows_mm_kernel,
        grid=(n // bm,),
        in_specs=[
            pl.BlockSpec((bm, k), lambda i: (i, 0)),
            pl.BlockSpec((k, m), lambda i: (0, 0)),
        ],
        out_specs=pl.BlockSpec((bm, m), lambda i: (i, 0)),
        out_shape=jax.ShapeDtypeStruct((n, m), out_dtype),
        compiler_params=pltpu.CompilerParams(
            dimension_semantics=("parallel",),
        ),
    )(x, w)


def _h1q_kernel(a_ref, p_ref, w23_ref, q_ref):
    h1 = jnp.maximum(
        jnp.dot(a_ref[...].astype(jnp.bfloat16), p_ref[...],
                preferred_element_type=jnp.float32), 0.0)
    q_ref[...] = jnp.dot(
        h1.astype(jnp.bfloat16), w23_ref[...],
        preferred_element_type=jnp.float32).astype(jnp.bfloat16)


def _h1q_mm(adj, p, w23):
    n = adj.shape[0]
    m = p.shape[1]
    mq = w23.shape[1]
    return pl.pallas_call(
        _h1q_kernel,
        grid=(n // _BM,),
        in_specs=[
            pl.BlockSpec((_BM, n), lambda i: (i, 0)),
            pl.BlockSpec((n, m), lambda i: (0, 0)),
            pl.BlockSpec((m, mq), lambda i: (0, 0)),
        ],
        out_specs=pl.BlockSpec((_BM, mq), lambda i: (i, 0)),
        out_shape=jax.ShapeDtypeStruct((n, mq), jnp.bfloat16),
        compiler_params=pltpu.CompilerParams(
            dimension_semantics=("parallel",),
        ),
    )(adj, p, w23)


def _tail_kernel(a_ref, q_ref, eps_ref, a1w_ref, a1b_ref, a2w_ref, a2b_ref,
                 mu_ref, lv_ref, z_ref, adv_ref):
    acc = jnp.dot(a_ref[...].astype(jnp.bfloat16), q_ref[...],
                  preferred_element_type=jnp.float32)
    mu = acc[:, :_H2]
    lv = acc[:, _H2:]
    z = eps_ref[...] * jnp.exp(lv) + mu
    mu_ref[...] = mu
    lv_ref[...] = lv
    z_ref[...] = z
    hidden = jnp.maximum(
        jnp.dot(z, a1w_ref[...], preferred_element_type=jnp.float32)
        + a1b_ref[...], 0.0)
    adv_ref[...] = (
        jnp.dot(hidden, a2w_ref[...], preferred_element_type=jnp.float32)
        + a2b_ref[...])


def _tail_mm(adj, q, eps, a1w, a1b, a2w, a2b):
    n = adj.shape[0]
    out_shapes = (
        jax.ShapeDtypeStruct((n, _H2), jnp.float32),
        jax.ShapeDtypeStruct((n, _H2), jnp.float32),
        jax.ShapeDtypeStruct((n, _H2), jnp.float32),
        jax.ShapeDtypeStruct((n, 1), jnp.float32),
    )
    out_spec = pl.BlockSpec((_BM, _H2), lambda i: (i, 0))
    return pl.pallas_call(
        _tail_kernel,
        grid=(n // _BM,),
        in_specs=[
            pl.BlockSpec((_BM, n), lambda i: (i, 0)),
            pl.BlockSpec((n, 2 * _H2), lambda i: (0, 0)),
            pl.BlockSpec((_BM, _H2), lambda i: (i, 0)),
            pl.BlockSpec((_H2, _H2), lambda i: (0, 0)),
            pl.BlockSpec((1, _H2), lambda i: (0, 0)),
            pl.BlockSpec((_H2, 1), lambda i: (0, 0)),
            pl.BlockSpec((1, 1), lambda i: (0, 0)),
        ],
        out_specs=(out_spec, out_spec, out_spec,
                   pl.BlockSpec((_BM, 1), lambda i: (i, 0))),
        out_shape=out_shapes,
        compiler_params=pltpu.CompilerParams(
            dimension_semantics=("parallel",),
        ),
    )(adj, q, eps, a1w, a1b, a2w, a2b)


def _recov_kernel(zi_ref, zt_ref, o_ref):
    o_ref[...] = jnp.dot(zi_ref[...].astype(jnp.bfloat16),
                         zt_ref[...].astype(jnp.bfloat16),
                         preferred_element_type=jnp.float32)


def _recov_mm(z, zt):
    n = z.shape[0]
    return pl.pallas_call(
        _recov_kernel,
        grid=(n // _BM,),
        in_specs=[
            pl.BlockSpec((_BM, _H2), lambda i: (i, 0)),
            pl.BlockSpec((_H2, n), lambda i: (0, 0)),
        ],
        out_specs=pl.BlockSpec((_BM, n), lambda i: (i, 0)),
        out_shape=jax.ShapeDtypeStruct((n, n), jnp.float32),
        compiler_params=pltpu.CompilerParams(
            dimension_semantics=("parallel",),
        ),
    )(z, zt)


_DEPTH = 8


def _link_body(recov_hbm, idx_hbm, out_hbm, idxs, vals, sem_g):
    wid = lax.axis_index("s") * _NC + lax.axis_index("c")

    pltpu.sync_copy(idx_hbm.at[wid], idxs)

    def chunk(c, _):
        pltpu.async_copy(recov_hbm.at[idxs.at[c]], vals.at[c], sem_g)

        @pl.when(c >= _DEPTH)
        def _():
            pltpu.make_async_copy(
                recov_hbm.at[idxs.at[0]], vals.at[0], sem_g).wait()

        return 0

    lax.fori_loop(0, _NCHUNK, chunk, 0)
    for _ in range(_DEPTH):
        pltpu.make_async_copy(recov_hbm.at[idxs.at[0]], vals.at[0], sem_g).wait()
    pltpu.sync_copy(vals, out_hbm.at[wid])


def _link_preds_sc(recov_flat, idxr):
    mesh = plsc.VectorSubcoreMesh(
        core_axis_name="c", subcore_axis_name="s",
        num_cores=_NC, num_subcores=_NS)
    k = pl.kernel(
        _link_body,
        out_type=jax.ShapeDtypeStruct((_NW, _NCHUNK, _CHUNK), jnp.float32),
        mesh=mesh,
        scratch_types=[
            pltpu.VMEM((_NCHUNK, _CHUNK), jnp.int32),
            pltpu.VMEM((_NCHUNK, _CHUNK), jnp.float32),
            pltpu.SemaphoreType.DMA,
        ],
        compiler_params=pltpu.CompilerParams(needs_layout_passes=False),
    )
    return k(recov_flat, idxr)


def kernel(feats, adj, edges, W1, W2, W3, A1w, A1b, A2w, A2b, eps):
    w23 = jnp.concatenate([W2, W3], axis=1).astype(jnp.bfloat16)
    p = _rows_mm(feats, W1, 2000, out_dtype=jnp.bfloat16)
    q = _h1q_mm(adj, p, w23)
    mu, logvar, z, adv_preds = _tail_mm(
        adj, q, eps, A1w, A1b.reshape(1, _H2), A2w, A2b.reshape(1, 1))

    recov = _recov_mm(z, z.T)

    pad = _EPAD - _E
    flat = edges[:, 0] * _N + edges[:, 1]
    flat = jnp.concatenate([flat, jnp.zeros((pad,), flat.dtype)])
    idxr = flat.astype(jnp.int32).reshape(_NW, _NCHUNK, _CHUNK)
    link = _link_preds_sc(recov.reshape(-1), idxr).reshape(-1)[:_E]

    return (recov, mu, logvar, link, adv_preds)
